# baseline (speedup 1.0000x reference)
import functools

import jax
import jax.numpy as jnp
from jax import lax
from jax.experimental import pallas as pl
from jax.experimental.pallas import tpu as pltpu
from jax.experimental.pallas import tpu_sc as plsc

NUM_EMB = 100000
DIM = 32
BATCH = 16384

CHUNK = 128


@functools.lru_cache(maxsize=None)
def _build_gather():
    info = plsc.get_sparse_core_info()
    nw = info.num_cores * info.num_subcores
    b_per_w = BATCH // nw
    nchunk = b_per_w // CHUNK
    mesh = plsc.VectorSubcoreMesh(core_axis_name="c", subcore_axis_name="s")

    @functools.partial(
        pl.kernel,
        mesh=mesh,
        out_type=jax.ShapeDtypeStruct((BATCH, DIM), jnp.float32),
        scratch_types=[
            pltpu.VMEM((b_per_w,), jnp.int32),
            pltpu.VMEM((b_per_w, DIM), jnp.float32),
            pltpu.SemaphoreType.DMA,
        ],
        compiler_params=pltpu.CompilerParams(use_tc_tiling_on_sc=False),
    )
    def gather(idx_hbm, table_hbm, out_hbm, idx_v, rows_v, sem):
        wid = lax.axis_index("s") * info.num_cores + lax.axis_index("c")
        base = wid * b_per_w
        pltpu.sync_copy(idx_hbm.at[pl.ds(base, b_per_w)], idx_v)
        copies = [
            pltpu.async_copy(
                table_hbm.at[idx_v.at[pl.ds(j * CHUNK, CHUNK)]],
                rows_v.at[pl.ds(j * CHUNK, CHUNK)],
                sem,
            )
            for j in range(nchunk)
        ]
        for cp in copies:
            cp.wait()
        pltpu.sync_copy(rows_v, out_hbm.at[pl.ds(base, b_per_w)])

    return gather


def kernel(x, table):
    return _build_gather()(x.astype(jnp.int32), table)


# speedup vs baseline: 2.2374x; 2.2374x over previous
import functools

import jax
import jax.numpy as jnp
from jax import lax
from jax.experimental import pallas as pl
from jax.experimental.pallas import tpu as pltpu
from jax.experimental.pallas import tpu_sc as plsc

NUM_EMB = 100000
DIM = 32
BATCH = 16384

CHUNK = 2048
LANES = 16


@functools.lru_cache(maxsize=None)
def _build_gather():
    info = plsc.get_sparse_core_info()
    nw = info.num_cores * info.num_subcores
    assert nw == DIM
    nchunk = BATCH // CHUNK
    mesh = plsc.VectorSubcoreMesh(core_axis_name="c", subcore_axis_name="s")

    @functools.partial(
        pl.kernel,
        mesh=mesh,
        out_type=jax.ShapeDtypeStruct((DIM, BATCH), jnp.float32),
        scratch_types=[
            pltpu.VMEM((NUM_EMB,), jnp.float32),
            pltpu.VMEM((BATCH,), jnp.int32),
            pltpu.VMEM((CHUNK,), jnp.float32),
            pltpu.SemaphoreType.DMA,
        ],
        compiler_params=pltpu.CompilerParams(needs_layout_passes=False),
    )
    def gather(idx_hbm, tablet_hbm, outt_hbm, row_v, idx_v, out_v, sem):
        c = lax.axis_index("s") * info.num_cores + lax.axis_index("c")
        cp_row = pltpu.async_copy(tablet_hbm.at[c], row_v, sem)
        pltpu.sync_copy(idx_hbm, idx_v)
        cp_row.wait()

        for ch in range(nchunk):
            base = ch * CHUNK

            def body(j, carry):
                iv = idx_v[pl.ds(base + j * LANES, LANES)]
                out_v[pl.ds(j * LANES, LANES)] = plsc.load_gather(row_v, [iv])
                return carry

            lax.fori_loop(0, CHUNK // LANES, body, 0)
            pltpu.sync_copy(out_v, outt_hbm.at[c, pl.ds(base, CHUNK)])

    return gather


def kernel(x, table):
    outt = _build_gather()(x.astype(jnp.int32), table.T)
    return outt.T


# speedup vs baseline: 2.2679x; 1.0136x over previous
import functools

import jax
import jax.numpy as jnp
from jax import lax
from jax.experimental import pallas as pl
from jax.experimental.pallas import tpu as pltpu
from jax.experimental.pallas import tpu_sc as plsc

NUM_EMB = 100000
DIM = 32
BATCH = 16384

QUARTER = BATCH // 4
LANES = 16
UNROLL = 8


@functools.lru_cache(maxsize=None)
def _build_gather():
    info = plsc.get_sparse_core_info()
    nw = info.num_cores * info.num_subcores
    assert nw == DIM
    mesh = plsc.VectorSubcoreMesh(core_axis_name="c", subcore_axis_name="s")

    @functools.partial(
        pl.kernel,
        mesh=mesh,
        out_type=jax.ShapeDtypeStruct((DIM, BATCH), jnp.float32),
        scratch_types=[
            pltpu.VMEM((NUM_EMB,), jnp.float32),
            pltpu.VMEM((BATCH,), jnp.int32),
            pltpu.VMEM((2, QUARTER), jnp.float32),
            pltpu.SemaphoreType.DMA,
            pltpu.SemaphoreType.DMA,
            pltpu.SemaphoreType.DMA,
        ],
        compiler_params=pltpu.CompilerParams(needs_layout_passes=False),
    )
    def gather(idx_hbm, tablet_hbm, outt_hbm, row_v, idx_v, out_v, sem_r,
               sem_a, sem_b):
        c = lax.axis_index("s") * info.num_cores + lax.axis_index("c")
        cp_row = pltpu.async_copy(tablet_hbm.at[c], row_v, sem_r)
        pltpu.sync_copy(idx_hbm, idx_v)
        cp_row.wait()

        step = LANES * UNROLL
        out_sems = (sem_a, sem_b)
        prev = None
        for h in range(4):
            base = h * QUARTER

            def body(j, carry):
                off = j * step
                for k in range(UNROLL):
                    iv = idx_v[pl.ds(base + off + k * LANES, LANES)]
                    out_v[h % 2, pl.ds(off + k * LANES, LANES)] = (
                        plsc.load_gather(row_v, [iv]))
                return carry

            lax.fori_loop(0, QUARTER // step, body, 0)
            if prev is not None:
                prev.wait()
            prev = pltpu.async_copy(
                out_v.at[h % 2], outt_hbm.at[c, pl.ds(base, QUARTER)],
                out_sems[h % 2])
        prev.wait()

    return gather


def kernel(x, table):
    outt = _build_gather()(x.astype(jnp.int32), table.T)
    return outt.T


# speedup vs baseline: 2.8759x; 1.2681x over previous
import functools

import jax
import jax.numpy as jnp
from jax import lax
from jax.experimental import pallas as pl
from jax.experimental.pallas import tpu as pltpu
from jax.experimental.pallas import tpu_sc as plsc

NUM_EMB = 100000
DIM = 32
BATCH = 16384

QUARTER = BATCH // 4
LANES = 16
UNROLL = 8


@functools.lru_cache(maxsize=None)
def _build_gather():
    info = plsc.get_sparse_core_info()
    nw = info.num_cores * info.num_subcores
    assert nw == DIM
    mesh = plsc.VectorSubcoreMesh(core_axis_name="c", subcore_axis_name="s")

    @functools.partial(
        pl.kernel,
        mesh=mesh,
        out_type=jax.ShapeDtypeStruct((DIM, BATCH), jnp.float32),
        scratch_types=[
            pltpu.VMEM((NUM_EMB,), jnp.float32),
            pltpu.VMEM((BATCH,), jnp.int32),
            pltpu.VMEM((2, QUARTER), jnp.float32),
            pltpu.SemaphoreType.DMA,
            pltpu.SemaphoreType.DMA,
            pltpu.SemaphoreType.DMA,
        ],
        compiler_params=pltpu.CompilerParams(needs_layout_passes=False),
    )
    def gather(idx_hbm, tablet_hbm, outt_hbm, row_v, idx_v, out_v, sem_r,
               sem_a, sem_b):
        c = lax.axis_index("s") * info.num_cores + lax.axis_index("c")
        cp_row = pltpu.async_copy(tablet_hbm.at[c], row_v, sem_r)
        pltpu.sync_copy(idx_hbm, idx_v)
        cp_row.wait()

        step = LANES * UNROLL
        out_sems = (sem_a, sem_b)
        prev = None
        for h in range(4):
            base = h * QUARTER

            @plsc.parallel_loop(0, QUARTER // LANES, unroll=UNROLL)
            def body(j):
                off = j * LANES
                iv = idx_v[pl.ds(base + off, LANES)]
                out_v[h % 2, pl.ds(off, LANES)] = (
                    plsc.load_gather(row_v, [iv]))
            if prev is not None:
                prev.wait()
            prev = pltpu.async_copy(
                out_v.at[h % 2], outt_hbm.at[c, pl.ds(base, QUARTER)],
                out_sems[h % 2])
        prev.wait()

    return gather


def kernel(x, table):
    outt = _build_gather()(x.astype(jnp.int32), table.T)
    return outt.T
